# baseline (speedup 1.0000x reference)
import jax
import jax.numpy as jnp
from jax import lax
from jax.experimental import pallas as pl
from jax.experimental.pallas import tpu as pltpu
from jax.experimental.pallas import tpu_sc as plsc

M, N, DIM = 128, 64, 256
MN = M * N
ALPHA = 0.3
SIGMA = 64.0
EPS = 1e-6
L = 16
NC, NS = 2, 16
NW = NC * NS
ROWS = MN // NW
CHUNK = ROWS // 2
CH = DIM // L
SCAN = 2 * ROWS

_mesh = plsc.VectorSubcoreMesh(core_axis_name="c", subcore_axis_name="s")

PCH = 64

_SCRATCH = [
    pltpu.VMEM((ROWS, DIM), jnp.float32),
    pltpu.VMEM((PCH, DIM), jnp.float32),
    pltpu.VMEM((PCH, DIM), jnp.float32),
    pltpu.VMEM((SCAN * L,), jnp.float32),
    pltpu.VMEM((DIM,), jnp.float32),
    pltpu.VMEM((DIM,), jnp.float32),
    pltpu.VMEM((L,), jnp.float32),
    pltpu.VMEM((L,), jnp.int32),
    pltpu.VMEM((2 * L,), jnp.float32),
    pltpu.VMEM((NS * L,), jnp.float32),
    pltpu.VMEM((NS * L,), jnp.int32),
    pltpu.SemaphoreType.DMA,
    pltpu.SemaphoreType.DMA,
    pltpu.SemaphoreType.DMA,
]


def _som_body(x_hbm, w_hbm, loc_hbm, out_hbm, exv_hbm, exi_hbm,
              obuf, pbuf0, pbuf1, dpart, xv, xe, recv, reci, lrb, av, ai,
              sema, semb, semo):
    del loc_hbm
    c = lax.axis_index("c")
    s = lax.axis_index("s")
    wid = c * NS + s
    pwid = (1 - c) * NS + s
    obase = wid * ROWS
    pbase = pwid * ROWS

    cpo = pltpu.async_copy(w_hbm.at[pl.ds(obase, ROWS)], obuf, semo)
    cp0 = pltpu.async_copy(w_hbm.at[pl.ds(pbase, PCH)], pbuf0, sema)
    cp1 = pltpu.async_copy(w_hbm.at[pl.ds(pbase + PCH, PCH)], pbuf1, semb)
    pltpu.sync_copy(x_hbm, xv)
    for ci in range(CH):
        xe[pl.ds(ci * L, L)] = xv[pl.ds(ci * L, L)] + EPS

    xes = [xe[pl.ds(ci * L, L)] for ci in range(CH)]
    iota = lax.iota(jnp.int32, L)
    siota = iota * SCAN

    def scan_chunk(buf, nrows, dbase):
        @plsc.parallel_loop(0, nrows, unroll=2)
        def _(r):
            a0 = jnp.zeros((L,), jnp.float32)
            a1 = jnp.zeros((L,), jnp.float32)
            a2 = jnp.zeros((L,), jnp.float32)
            a3 = jnp.zeros((L,), jnp.float32)
            for ci in range(0, CH, 4):
                d0 = xes[ci] - buf[r, pl.ds(ci * L, L)]
                d1 = xes[ci + 1] - buf[r, pl.ds((ci + 1) * L, L)]
                d2 = xes[ci + 2] - buf[r, pl.ds((ci + 2) * L, L)]
                d3 = xes[ci + 3] - buf[r, pl.ds((ci + 3) * L, L)]
                a0 = a0 + d0 * d0
                a1 = a1 + d1 * d1
                a2 = a2 + d2 * d2
                a3 = a3 + d3 * d3
            plsc.store_scatter(dpart, [siota + (dbase + r)],
                               (a0 + a1) + (a2 + a3))

    cp0.wait()
    scan_chunk(pbuf0, PCH, 0)
    cp2 = pltpu.async_copy(w_hbm.at[pl.ds(pbase + 2 * PCH, PCH)], pbuf0, sema)
    cp1.wait()
    scan_chunk(pbuf1, PCH, PCH)
    cp3 = pltpu.async_copy(w_hbm.at[pl.ds(pbase + 3 * PCH, PCH)], pbuf1, semb)
    cp2.wait()
    scan_chunk(pbuf0, PCH, 2 * PCH)
    cp3.wait()
    scan_chunk(pbuf1, PCH, 3 * PCH)
    cpo.wait()
    scan_chunk(obuf, ROWS, 4 * PCH)


    def red_body(g, cr):
        minv, mini = cr
        lrows = g * L + iota
        grows = jnp.where(lrows < SCAN // 2, pbase + lrows,
                          obase + (lrows - SCAN // 2))
        gs = [dpart[pl.ds(j * SCAN + g * L, L)] for j in range(L)]
        while len(gs) > 1:
            gs = [gs[k] + gs[k + 1] for k in range(0, len(gs), 2)]
        t = gs[0]
        lt = (t < minv) | ((t == minv) & (grows < mini))
        return (jnp.where(lt, t, minv), jnp.where(lt, grows, mini))

    minv, mini = lax.fori_loop(
        0, SCAN // L, red_body,
        (jnp.full((L,), 3.0e38, jnp.float32), jnp.zeros((L,), jnp.int32)))
    bestv = jnp.min(minv)
    besti = jnp.min(jnp.where(minv == bestv, mini, jnp.int32(2**31 - 1)))

    recv[:] = jnp.full((L,), bestv, jnp.float32)
    reci[:] = jnp.full((L,), besti, jnp.int32)
    pltpu.sync_copy(recv, exv_hbm.at[c, pl.ds(s * L, L)])
    pltpu.sync_copy(reci, exi_hbm.at[c, pl.ds(s * L, L)])
    plsc.subcore_barrier()
    pltpu.sync_copy(exv_hbm.at[c], av)
    pltpu.sync_copy(exi_hbm.at[c], ai)
    m = av[pl.ds(0, L)]
    mi = ai[pl.ds(0, L)]
    for t in range(1, NS):
        v = av[pl.ds(t * L, L)]
        i = ai[pl.ds(t * L, L)]
        lt = (v < m) | ((v == m) & (i < mi))
        m = jnp.where(lt, v, m)
        mi = jnp.where(lt, i, mi)
    bx = mi % M
    by = mi // M

    neg_inv_s2 = jnp.float32(-1.0 / (SIGMA * SIGMA))

    xs = [xv[pl.ds(ci * L, L)] for ci in range(CH)]

    def update_half(half):
        def grp_body(g, _):
            rows = obase + half * CHUNK + g * L + iota
            dx = (rows % M - bx).astype(jnp.float32)
            dy = (rows // M - by).astype(jnp.float32)
            lrb[pl.ds(0, L)] = ALPHA * jnp.exp((dx * dx + dy * dy) * neg_inv_s2)

            @plsc.parallel_loop(0, L // 2)
            def _(p):
                la = plsc.load_gather(lrb, [jnp.full((L,), 2 * p, jnp.int32)])
                lb = plsc.load_gather(lrb, [jnp.full((L,), 2 * p + 1,
                                                     jnp.int32)])
                ra = half * CHUNK + g * L + 2 * p
                rb = ra + 1
                for ci in range(CH):
                    wa = obuf[ra, pl.ds(ci * L, L)]
                    wb = obuf[rb, pl.ds(ci * L, L)]
                    obuf[ra, pl.ds(ci * L, L)] = wa + la * (xs[ci] - wa)
                    obuf[rb, pl.ds(ci * L, L)] = wb + lb * (xs[ci] - wb)

            return 0

        lax.fori_loop(0, CHUNK // L, grp_body, 0)

    update_half(0)
    cw0 = pltpu.async_copy(obuf.at[pl.ds(0, CHUNK)],
                           out_hbm.at[pl.ds(obase, CHUNK)], sema)
    update_half(1)
    cw1 = pltpu.async_copy(obuf.at[pl.ds(CHUNK, CHUNK)],
                           out_hbm.at[pl.ds(obase + CHUNK, CHUNK)], semb)
    cw0.wait()
    cw1.wait()


_som_update = pl.kernel(
    _som_body,
    out_type=(
        jax.ShapeDtypeStruct((MN, DIM), jnp.float32),
        jax.ShapeDtypeStruct((NC, NS * L), jnp.float32),
        jax.ShapeDtypeStruct((NC, NS * L), jnp.int32),
    ),
    mesh=_mesh,
    scratch_types=_SCRATCH,
    compiler_params=pltpu.CompilerParams(needs_layout_passes=False),
)


def kernel(x, weights, locations):
    return _som_update(x, weights, locations)[0]


# speedup vs baseline: 1.0527x; 1.0527x over previous
import jax
import jax.numpy as jnp
from jax import lax
from jax.experimental import pallas as pl
from jax.experimental.pallas import tpu as pltpu
from jax.experimental.pallas import tpu_sc as plsc

M, N, DIM = 128, 64, 256
MN = M * N
ALPHA = 0.3
SIGMA = 64.0
EPS = 1e-6
L = 16
NC, NS = 2, 16
NW = NC * NS
ROWS = MN // NW
CHUNK = ROWS // 2
CH = DIM // L
SCAN = 2 * ROWS

_mesh = plsc.VectorSubcoreMesh(core_axis_name="c", subcore_axis_name="s")

PCH = 64

_SCRATCH = [
    pltpu.VMEM((ROWS, DIM), jnp.float32),
    pltpu.VMEM((PCH, DIM), jnp.float32),
    pltpu.VMEM((PCH, DIM), jnp.float32),
    pltpu.VMEM((SCAN * L,), jnp.float32),
    pltpu.VMEM((DIM,), jnp.float32),
    pltpu.VMEM((DIM,), jnp.float32),
    pltpu.VMEM((L,), jnp.float32),
    pltpu.VMEM((L,), jnp.int32),
    pltpu.VMEM((2 * L,), jnp.float32),
    pltpu.VMEM((NS * L,), jnp.float32),
    pltpu.VMEM((NS * L,), jnp.int32),
    pltpu.SemaphoreType.DMA,
    pltpu.SemaphoreType.DMA,
    pltpu.SemaphoreType.DMA,
]


def _som_body(x_hbm, w_hbm, loc_hbm, out_hbm, exv_hbm, exi_hbm,
              obuf, pbuf0, pbuf1, dpart, xv, xe, recv, reci, lrb, av, ai,
              sema, semb, semo):
    del loc_hbm
    c = lax.axis_index("c")
    s = lax.axis_index("s")
    wid = c * NS + s
    pwid = (1 - c) * NS + s
    obase = wid * ROWS
    pbase = pwid * ROWS

    cpo = pltpu.async_copy(w_hbm.at[pl.ds(obase, ROWS)], obuf, semo)
    cp0 = pltpu.async_copy(w_hbm.at[pl.ds(pbase, PCH)], pbuf0, sema)
    cp1 = pltpu.async_copy(w_hbm.at[pl.ds(pbase + PCH, PCH)], pbuf1, semb)
    pltpu.sync_copy(x_hbm, xv)
    for ci in range(CH):
        xe[pl.ds(ci * L, L)] = xv[pl.ds(ci * L, L)] + EPS

    xes = [xe[pl.ds(ci * L, L)] for ci in range(CH)]
    iota = lax.iota(jnp.int32, L)
    siota = iota * SCAN

    def scan_chunk(buf, nrows, dbase):
        @plsc.parallel_loop(0, nrows, unroll=2)
        def _(r):
            a0 = jnp.zeros((L,), jnp.float32)
            a1 = jnp.zeros((L,), jnp.float32)
            a2 = jnp.zeros((L,), jnp.float32)
            a3 = jnp.zeros((L,), jnp.float32)
            for ci in range(0, CH, 4):
                d0 = xes[ci] - buf[r, pl.ds(ci * L, L)]
                d1 = xes[ci + 1] - buf[r, pl.ds((ci + 1) * L, L)]
                d2 = xes[ci + 2] - buf[r, pl.ds((ci + 2) * L, L)]
                d3 = xes[ci + 3] - buf[r, pl.ds((ci + 3) * L, L)]
                a0 = a0 + d0 * d0
                a1 = a1 + d1 * d1
                a2 = a2 + d2 * d2
                a3 = a3 + d3 * d3
            dpart[pl.ds((dbase + r) * L, L)] = (a0 + a1) + (a2 + a3)

    cp0.wait()
    scan_chunk(pbuf0, PCH, 0)
    cp2 = pltpu.async_copy(w_hbm.at[pl.ds(pbase + 2 * PCH, PCH)], pbuf0, sema)
    cp1.wait()
    scan_chunk(pbuf1, PCH, PCH)
    cp3 = pltpu.async_copy(w_hbm.at[pl.ds(pbase + 3 * PCH, PCH)], pbuf1, semb)
    cp2.wait()
    scan_chunk(pbuf0, PCH, 2 * PCH)
    cp3.wait()
    scan_chunk(pbuf1, PCH, 3 * PCH)
    cpo.wait()
    scan_chunk(obuf, ROWS, 4 * PCH)


    def red_body(g, cr):
        minv, mini = cr
        lrows = g * L + iota
        grows = jnp.where(lrows < SCAN // 2, pbase + lrows,
                          obase + (lrows - SCAN // 2))
        fbase = lrows * L
        gs = [plsc.load_gather(dpart, [fbase + j]) for j in range(L)]
        while len(gs) > 1:
            gs = [gs[k] + gs[k + 1] for k in range(0, len(gs), 2)]
        t = gs[0]
        lt = (t < minv) | ((t == minv) & (grows < mini))
        return (jnp.where(lt, t, minv), jnp.where(lt, grows, mini))

    minv, mini = lax.fori_loop(
        0, SCAN // L, red_body,
        (jnp.full((L,), 3.0e38, jnp.float32), jnp.zeros((L,), jnp.int32)))
    bestv = jnp.min(minv)
    besti = jnp.min(jnp.where(minv == bestv, mini, jnp.int32(2**31 - 1)))

    recv[:] = jnp.full((L,), bestv, jnp.float32)
    reci[:] = jnp.full((L,), besti, jnp.int32)
    pltpu.sync_copy(recv, exv_hbm.at[c, pl.ds(s * L, L)])
    pltpu.sync_copy(reci, exi_hbm.at[c, pl.ds(s * L, L)])
    plsc.subcore_barrier()
    pltpu.sync_copy(exv_hbm.at[c], av)
    pltpu.sync_copy(exi_hbm.at[c], ai)
    m = av[pl.ds(0, L)]
    mi = ai[pl.ds(0, L)]
    for t in range(1, NS):
        v = av[pl.ds(t * L, L)]
        i = ai[pl.ds(t * L, L)]
        lt = (v < m) | ((v == m) & (i < mi))
        m = jnp.where(lt, v, m)
        mi = jnp.where(lt, i, mi)
    bx = mi % M
    by = mi // M

    neg_inv_s2 = jnp.float32(-1.0 / (SIGMA * SIGMA))

    xs = [xv[pl.ds(ci * L, L)] for ci in range(CH)]

    def update_half(half):
        def grp_body(g, _):
            rows = obase + half * CHUNK + g * L + iota
            dx = (rows % M - bx).astype(jnp.float32)
            dy = (rows // M - by).astype(jnp.float32)
            lrb[pl.ds(0, L)] = ALPHA * jnp.exp((dx * dx + dy * dy) * neg_inv_s2)

            @plsc.parallel_loop(0, L // 2)
            def _(p):
                la = plsc.load_gather(lrb, [jnp.full((L,), 2 * p, jnp.int32)])
                lb = plsc.load_gather(lrb, [jnp.full((L,), 2 * p + 1,
                                                     jnp.int32)])
                ra = half * CHUNK + g * L + 2 * p
                rb = ra + 1
                for ci in range(CH):
                    wa = obuf[ra, pl.ds(ci * L, L)]
                    wb = obuf[rb, pl.ds(ci * L, L)]
                    obuf[ra, pl.ds(ci * L, L)] = wa + la * (xs[ci] - wa)
                    obuf[rb, pl.ds(ci * L, L)] = wb + lb * (xs[ci] - wb)

            return 0

        lax.fori_loop(0, CHUNK // L, grp_body, 0)

    update_half(0)
    cw0 = pltpu.async_copy(obuf.at[pl.ds(0, CHUNK)],
                           out_hbm.at[pl.ds(obase, CHUNK)], sema)
    update_half(1)
    cw1 = pltpu.async_copy(obuf.at[pl.ds(CHUNK, CHUNK)],
                           out_hbm.at[pl.ds(obase + CHUNK, CHUNK)], semb)
    cw0.wait()
    cw1.wait()


_som_update = pl.kernel(
    _som_body,
    out_type=(
        jax.ShapeDtypeStruct((MN, DIM), jnp.float32),
        jax.ShapeDtypeStruct((NC, NS * L), jnp.float32),
        jax.ShapeDtypeStruct((NC, NS * L), jnp.int32),
    ),
    mesh=_mesh,
    scratch_types=_SCRATCH,
    compiler_params=pltpu.CompilerParams(needs_layout_passes=False),
)


def kernel(x, weights, locations):
    return _som_update(x, weights, locations)[0]


# speedup vs baseline: 1.0544x; 1.0017x over previous
import jax
import jax.numpy as jnp
from jax import lax
from jax.experimental import pallas as pl
from jax.experimental.pallas import tpu as pltpu
from jax.experimental.pallas import tpu_sc as plsc

M, N, DIM = 128, 64, 256
MN = M * N
ALPHA = 0.3
SIGMA = 64.0
EPS = 1e-6
L = 16
NC, NS = 2, 16
NW = NC * NS
ROWS = MN // NW
CHUNK = ROWS // 2
CH = DIM // L
SCAN = 2 * ROWS

_mesh = plsc.VectorSubcoreMesh(core_axis_name="c", subcore_axis_name="s")

PCH = 64

_SCRATCH = [
    pltpu.VMEM((ROWS, DIM), jnp.float32),
    pltpu.VMEM((PCH, DIM), jnp.float32),
    pltpu.VMEM((PCH, DIM), jnp.float32),
    pltpu.VMEM((SCAN * L,), jnp.float32),
    pltpu.VMEM((DIM,), jnp.float32),
    pltpu.VMEM((DIM,), jnp.float32),
    pltpu.VMEM((L,), jnp.float32),
    pltpu.VMEM((L,), jnp.int32),
    pltpu.VMEM((2 * L,), jnp.float32),
    pltpu.VMEM((NS * L,), jnp.float32),
    pltpu.VMEM((NS * L,), jnp.int32),
    pltpu.SemaphoreType.DMA,
    pltpu.SemaphoreType.DMA,
    pltpu.SemaphoreType.DMA,
]


def _som_body(x_hbm, w_hbm, loc_hbm, out_hbm, exv_hbm, exi_hbm,
              obuf, pbuf0, pbuf1, dpart, xv, xe, recv, reci, lrb, av, ai,
              sema, semb, semo):
    del loc_hbm
    c = lax.axis_index("c")
    s = lax.axis_index("s")
    wid = c * NS + s
    pwid = (1 - c) * NS + s
    obase = wid * ROWS
    pbase = pwid * ROWS

    cpo = pltpu.async_copy(w_hbm.at[pl.ds(obase, ROWS)], obuf, semo)
    cp0 = pltpu.async_copy(w_hbm.at[pl.ds(pbase, PCH)], pbuf0, sema)
    cp1 = pltpu.async_copy(w_hbm.at[pl.ds(pbase + PCH, PCH)], pbuf1, semb)
    pltpu.sync_copy(x_hbm, xv)
    for ci in range(CH):
        xe[pl.ds(ci * L, L)] = xv[pl.ds(ci * L, L)] + EPS

    xes = [xe[pl.ds(ci * L, L)] for ci in range(CH)]
    iota = lax.iota(jnp.int32, L)

    def scan_chunk(buf, nrows, dbase):
        @plsc.parallel_loop(0, nrows, unroll=2)
        def _(r):
            a0 = jnp.zeros((L,), jnp.float32)
            a1 = jnp.zeros((L,), jnp.float32)
            a2 = jnp.zeros((L,), jnp.float32)
            a3 = jnp.zeros((L,), jnp.float32)
            for ci in range(0, CH, 4):
                d0 = xes[ci] - buf[r, pl.ds(ci * L, L)]
                d1 = xes[ci + 1] - buf[r, pl.ds((ci + 1) * L, L)]
                d2 = xes[ci + 2] - buf[r, pl.ds((ci + 2) * L, L)]
                d3 = xes[ci + 3] - buf[r, pl.ds((ci + 3) * L, L)]
                a0 = a0 + d0 * d0
                a1 = a1 + d1 * d1
                a2 = a2 + d2 * d2
                a3 = a3 + d3 * d3
            dpart[pl.ds((dbase + r) * L, L)] = (a0 + a1) + (a2 + a3)

    cp0.wait()
    scan_chunk(pbuf0, PCH, 0)
    cp2 = pltpu.async_copy(w_hbm.at[pl.ds(pbase + 2 * PCH, PCH)], pbuf0, sema)
    cp1.wait()
    scan_chunk(pbuf1, PCH, PCH)
    cp3 = pltpu.async_copy(w_hbm.at[pl.ds(pbase + 3 * PCH, PCH)], pbuf1, semb)
    cp2.wait()
    scan_chunk(pbuf0, PCH, 2 * PCH)
    cp3.wait()
    scan_chunk(pbuf1, PCH, 3 * PCH)
    cpo.wait()
    scan_chunk(obuf, ROWS, 4 * PCH)


    def red_body(g, cr):
        minv, mini = cr
        lrows = g * L + iota
        grows = jnp.where(lrows < SCAN // 2, pbase + lrows,
                          obase + (lrows - SCAN // 2))
        fbase = lrows * L
        gs = [plsc.load_gather(dpart, [fbase + j]) for j in range(L)]
        while len(gs) > 1:
            gs = [gs[k] + gs[k + 1] for k in range(0, len(gs), 2)]
        t = gs[0]
        lt = (t < minv) | ((t == minv) & (grows < mini))
        return (jnp.where(lt, t, minv), jnp.where(lt, grows, mini))

    minv, mini = lax.fori_loop(
        0, SCAN // L, red_body,
        (jnp.full((L,), 3.0e38, jnp.float32), jnp.zeros((L,), jnp.int32)))
    bestv = jnp.min(minv)
    besti = jnp.min(jnp.where(minv == bestv, mini, jnp.int32(2**31 - 1)))

    recv[:] = jnp.full((L,), bestv, jnp.float32)
    reci[:] = jnp.full((L,), besti, jnp.int32)
    pltpu.sync_copy(recv, exv_hbm.at[c, pl.ds(s * L, L)])
    pltpu.sync_copy(reci, exi_hbm.at[c, pl.ds(s * L, L)])
    plsc.subcore_barrier()
    pltpu.sync_copy(exv_hbm.at[c], av)
    pltpu.sync_copy(exi_hbm.at[c], ai)
    m = av[pl.ds(0, L)]
    mi = ai[pl.ds(0, L)]
    for t in range(1, NS):
        v = av[pl.ds(t * L, L)]
        i = ai[pl.ds(t * L, L)]
        lt = (v < m) | ((v == m) & (i < mi))
        m = jnp.where(lt, v, m)
        mi = jnp.where(lt, i, mi)
    bx = mi % M
    by = mi // M

    neg_inv_s2 = jnp.float32(-1.0 / (SIGMA * SIGMA))

    xs = [xv[pl.ds(ci * L, L)] for ci in range(CH)]

    def update_half(half):
        def grp_body(g, _):
            rows = obase + half * CHUNK + g * L + iota
            dx = (rows % M - bx).astype(jnp.float32)
            dy = (rows // M - by).astype(jnp.float32)
            lrb[pl.ds(0, L)] = ALPHA * jnp.exp((dx * dx + dy * dy) * neg_inv_s2)

            @plsc.parallel_loop(0, L // 2)
            def _(p):
                la = plsc.load_gather(lrb, [jnp.full((L,), 2 * p, jnp.int32)])
                lb = plsc.load_gather(lrb, [jnp.full((L,), 2 * p + 1,
                                                     jnp.int32)])
                ra = half * CHUNK + g * L + 2 * p
                rb = ra + 1
                for ci in range(CH):
                    wa = obuf[ra, pl.ds(ci * L, L)]
                    wb = obuf[rb, pl.ds(ci * L, L)]
                    obuf[ra, pl.ds(ci * L, L)] = wa + la * (xs[ci] - wa)
                    obuf[rb, pl.ds(ci * L, L)] = wb + lb * (xs[ci] - wb)

            return 0

        lax.fori_loop(0, CHUNK // L, grp_body, 0)

    update_half(0)
    cw0 = pltpu.async_copy(obuf.at[pl.ds(0, CHUNK)],
                           out_hbm.at[pl.ds(obase, CHUNK)], sema)
    update_half(1)
    cw1 = pltpu.async_copy(obuf.at[pl.ds(CHUNK, CHUNK)],
                           out_hbm.at[pl.ds(obase + CHUNK, CHUNK)], semb)
    cw0.wait()
    cw1.wait()


_som_update = pl.kernel(
    _som_body,
    out_type=(
        jax.ShapeDtypeStruct((MN, DIM), jnp.float32),
        jax.ShapeDtypeStruct((NC, NS * L), jnp.float32),
        jax.ShapeDtypeStruct((NC, NS * L), jnp.int32),
    ),
    mesh=_mesh,
    scratch_types=_SCRATCH,
    compiler_params=pltpu.CompilerParams(needs_layout_passes=False),
)


def kernel(x, weights, locations):
    return _som_update(x, weights, locations)[0]
